# baseline (speedup 1.0000x reference)
import jax
import jax.numpy as jnp
from jax import lax
from jax.experimental import pallas as pl
from jax.experimental.pallas import tpu as pltpu
from jax.experimental.pallas import tpu_sc as plsc

N = 10000
D = 256
E = 160000
L = 16
NC = 2
NS = 16
NW = NC * NS
C = 128
NCHUNKS = E // C
NCW = (NCHUNKS + NW - 1) // NW
JB = D // 32


def _normalize_body(z_ref, out_ref):
    z = z_ref[...]
    ss = jnp.sum(z * z, axis=1, keepdims=True)
    inv = 1.0 / jnp.maximum(jnp.sqrt(ss), 1e-8)
    zn = (z * inv).astype(jnp.bfloat16)
    lo = jax.lax.bitcast_convert_type(
        zn[:, : D // 2], jnp.uint16).astype(jnp.uint32)
    hi = jax.lax.bitcast_convert_type(
        zn[:, D // 2:], jnp.uint16).astype(jnp.uint32)
    w = lo | (hi << 16)
    out_ref[...] = jax.lax.bitcast_convert_type(w, jnp.int32)


def _normalize(z):
    return pl.pallas_call(
        _normalize_body,
        out_shape=jax.ShapeDtypeStruct((N, D // 2), jnp.int32),
    )(z)


def _sc_body(zn_hbm, src_hbm, dst_hbm, out_hbm,
             i0, i1, a0, a1, b0, b1, out_v, sem0, sem1):
    wid = lax.axis_index("s") * NC + lax.axis_index("c")
    bufs = ((i0, a0, b0, sem0), (i1, a1, b1, sem1))

    def chunk_base(k):
        ci = jnp.minimum(wid + k * NW, NCHUNKS - 1)
        return ci * C

    def issue(k, b):
        iv, av, bv, sem = bufs[b]
        cb = chunk_base(k)
        pltpu.sync_copy(src_hbm.at[pl.ds(cb, C)], iv.at[pl.ds(0, C)])
        pltpu.sync_copy(dst_hbm.at[pl.ds(cb, C)], iv.at[pl.ds(C, C)])
        pltpu.async_copy(zn_hbm.at[iv.at[pl.ds(0, C)]], av, sem)
        pltpu.async_copy(zn_hbm.at[iv.at[pl.ds(C, C)]], bv, sem)

    def wait(b):
        iv, av, bv, sem = bufs[b]
        pltpu.make_async_copy(zn_hbm.at[iv.at[pl.ds(0, C)]], av, sem).wait()
        pltpu.make_async_copy(zn_hbm.at[iv.at[pl.ds(C, C)]], bv, sem).wait()

    lanes = lax.iota(jnp.int32, L)

    def compute(k, b):
        iv, av, bv, sem = bufs[b]
        cb = chunk_base(k)

        for g in range(C // L):
            rows = lanes + g * L

            def dim_body(t, accs):
                a0, a1 = accs
                for u in range(2):
                    d = t * 2 + u
                    cols = lanes * 0 + d
                    va = plsc.bitcast(
                        plsc.load_gather(av, [rows, cols]), jnp.bfloat16)
                    vb = plsc.bitcast(
                        plsc.load_gather(bv, [rows, cols]), jnp.bfloat16)
                    pm = va * vb
                    m_lo, m_hi = plsc.unpack(
                        pm, format=plsc.PackFormat.INTERLEAVED)
                    if u == 0:
                        a0 = a0 + (m_lo + m_hi)
                    else:
                        a1 = a1 + (m_lo + m_hi)
                return a0, a1

            zero = lanes * jnp.float32(0.0)
            a0, a1 = lax.fori_loop(0, (D // 2) // 2, dim_body, (zero, zero))
            acc = a0 + a1
            sig = 1.0 / (1.0 + jnp.exp(-acc))
            out_v[pl.ds(g * L, L)] = sig

        pltpu.sync_copy(out_v, out_hbm.at[pl.ds(cb, C)])

    issue(0, 0)

    def step(i, _):
        k0 = 2 * i
        k1 = 2 * i + 1
        wait(0)
        issue(k1, 1)
        compute(k0, 0)
        wait(1)

        @pl.when(k1 < NCW - 1)
        def _():
            issue(k1 + 1, 0)

        compute(k1, 1)
        return 0

    lax.fori_loop(0, NCW // 2, step, 0)


def _sc_decode(zn, src, dst):
    mesh = plsc.VectorSubcoreMesh(core_axis_name="c", subcore_axis_name="s")
    return pl.kernel(
        _sc_body,
        out_type=jax.ShapeDtypeStruct((E,), jnp.float32),
        mesh=mesh,
        scratch_types=[
            pltpu.VMEM((2 * C,), jnp.int32),
            pltpu.VMEM((2 * C,), jnp.int32),
            pltpu.VMEM((C, D // 2), jnp.int32),
            pltpu.VMEM((C, D // 2), jnp.int32),
            pltpu.VMEM((C, D // 2), jnp.int32),
            pltpu.VMEM((C, D // 2), jnp.int32),
            pltpu.VMEM((C,), jnp.float32),
            pltpu.SemaphoreType.DMA,
            pltpu.SemaphoreType.DMA,
        ],
        compiler_params=pltpu.CompilerParams(needs_layout_passes=False),
    )(zn, src, dst)


def kernel(z, edge_index):
    src = edge_index[0].astype(jnp.int32)
    dst = edge_index[1].astype(jnp.int32)
    zn = _normalize(z)
    return _sc_decode(zn, src, dst)


# speedup vs baseline: 4.4938x; 4.4938x over previous
import jax
import jax.numpy as jnp
from jax import lax
from jax.experimental import pallas as pl
from jax.experimental.pallas import tpu as pltpu
from jax.experimental.pallas import tpu_sc as plsc

N = 10000
D = 256
E = 160000
L = 16
NC = 2
NS = 16
NW = NC * NS
C = 128
NCHUNKS = E // C
NCW = (NCHUNKS + NW - 1) // NW
JB = D // 32


def _normalize_body(z_ref, out_ref):
    z = z_ref[...]
    ss = jnp.sum(z * z, axis=1, keepdims=True)
    inv = 1.0 / jnp.maximum(jnp.sqrt(ss), 1e-8)
    zn = (z * inv).astype(jnp.bfloat16)
    lo = jax.lax.bitcast_convert_type(
        zn[:, : D // 2], jnp.uint16).astype(jnp.uint32)
    hi = jax.lax.bitcast_convert_type(
        zn[:, D // 2:], jnp.uint16).astype(jnp.uint32)
    w = lo | (hi << 16)
    out_ref[...] = jax.lax.bitcast_convert_type(w, jnp.int32)


def _normalize(z):
    return pl.pallas_call(
        _normalize_body,
        out_shape=jax.ShapeDtypeStruct((N, D // 2), jnp.int32),
    )(z)


def _sc_body(zn_hbm, src_hbm, dst_hbm, out_hbm,
             i0, i1, a0, a1, b0, b1, p_v, out_v, sem0, sem1):
    wid = lax.axis_index("s") * NC + lax.axis_index("c")
    bufs = ((i0, a0, b0, sem0), (i1, a1, b1, sem1))

    def chunk_base(k):
        ci = jnp.minimum(wid + k * NW, NCHUNKS - 1)
        return ci * C

    def issue(k, b):
        iv, av, bv, sem = bufs[b]
        cb = chunk_base(k)
        pltpu.sync_copy(src_hbm.at[pl.ds(cb, C)], iv.at[pl.ds(0, C)])
        pltpu.sync_copy(dst_hbm.at[pl.ds(cb, C)], iv.at[pl.ds(C, C)])
        pltpu.async_copy(zn_hbm.at[iv.at[pl.ds(0, C)]], av, sem)
        pltpu.async_copy(zn_hbm.at[iv.at[pl.ds(C, C)]], bv, sem)

    def wait(b):
        iv, av, bv, sem = bufs[b]
        pltpu.make_async_copy(zn_hbm.at[iv.at[pl.ds(0, C)]], av, sem).wait()
        pltpu.make_async_copy(zn_hbm.at[iv.at[pl.ds(C, C)]], bv, sem).wait()

    lanes = lax.iota(jnp.int32, L)

    def compute(k, b):
        iv, av, bv, sem = bufs[b]
        cb = chunk_base(k)

        @plsc.parallel_loop(0, C, unroll=8)
        def edge_body(e):
            prods = []
            for j in range(JB):
                va = plsc.bitcast(av[e, pl.ds(j * L, L)], jnp.bfloat16)
                vb = plsc.bitcast(bv[e, pl.ds(j * L, L)], jnp.bfloat16)
                pm = va * vb
                m_lo, m_hi = plsc.unpack(pm, format=plsc.PackFormat.INTERLEAVED)
                prods.append(m_lo + m_hi)
            while len(prods) > 1:
                prods = [a + b for a, b in zip(prods[::2], prods[1::2])]
            plsc.store_scatter(p_v, [lanes + e * (L + 1)], prods[0])

        for g in range(C // L):
            base_idx = (lanes + g * L) * (L + 1)
            acc = plsc.load_gather(p_v, [base_idx])
            for d in range(1, L):
                acc = acc + plsc.load_gather(p_v, [base_idx + d])
            sig = 1.0 / (1.0 + jnp.exp(-acc))
            out_v[pl.ds(g * L, L)] = sig

        pltpu.sync_copy(out_v, out_hbm.at[pl.ds(cb, C)])

    issue(0, 0)

    def step(i, _):
        k0 = 2 * i
        k1 = 2 * i + 1
        wait(0)
        issue(k1, 1)
        compute(k0, 0)
        wait(1)

        @pl.when(k1 < NCW - 1)
        def _():
            issue(k1 + 1, 0)

        compute(k1, 1)
        return 0

    lax.fori_loop(0, NCW // 2, step, 0)


def _sc_decode(zn, src, dst):
    mesh = plsc.VectorSubcoreMesh(core_axis_name="c", subcore_axis_name="s")
    return pl.kernel(
        _sc_body,
        out_type=jax.ShapeDtypeStruct((E,), jnp.float32),
        mesh=mesh,
        scratch_types=[
            pltpu.VMEM((2 * C,), jnp.int32),
            pltpu.VMEM((2 * C,), jnp.int32),
            pltpu.VMEM((C, D // 2), jnp.int32),
            pltpu.VMEM((C, D // 2), jnp.int32),
            pltpu.VMEM((C, D // 2), jnp.int32),
            pltpu.VMEM((C, D // 2), jnp.int32),
            pltpu.VMEM((C * (L + 1),), jnp.float32),
            pltpu.VMEM((C,), jnp.float32),
            pltpu.SemaphoreType.DMA,
            pltpu.SemaphoreType.DMA,
        ],
        compiler_params=pltpu.CompilerParams(needs_layout_passes=False),
    )(zn, src, dst)


def kernel(z, edge_index):
    src = edge_index[0].astype(jnp.int32)
    dst = edge_index[1].astype(jnp.int32)
    zn = _normalize(z)
    return _sc_decode(zn, src, dst)
